# baseline (speedup 1.0000x reference)
import functools

import jax
import jax.numpy as jnp
from jax import lax
from jax.experimental import pallas as pl
from jax.experimental.pallas import tpu as pltpu
from jax.experimental.pallas import tpu_sc as plsc

S = 512
D = 128
T = 2 * S - 1
NC = 2
NS = 16
NW = NC * NS
RW = S // NW
W = 16
NCH = S // W
WIN = W + RW
L = 16


def _win_lo(i0, c):
    return pl.multiple_of(i0 + (S - 1) - c * W - (W - 1), W)


def _sc_body(emb_hbm, x_hbm, out_hbm,
             win0, win1, xb0, xb1, res0, res1,
             csem0, csem1, ssem0, ssem1):
    wid = lax.axis_index("s") * NC + lax.axis_index("c")
    i0 = wid * RW
    wins = (win0, win1)
    xbs = (xb0, xb1)
    ress = (res0, res1)
    csems = (csem0, csem1)
    ssems = (ssem0, ssem1)

    def issue_copies(c, b):
        pltpu.make_async_copy(
            emb_hbm.at[pl.ds(_win_lo(i0, c), WIN)], wins[b], csems[b]).start()
        pltpu.make_async_copy(
            x_hbm.at[pl.ds(pl.multiple_of(c * W, W), W)], xbs[b], csems[b]).start()

    def wait_copies(c, b):
        pltpu.make_async_copy(
            emb_hbm.at[pl.ds(_win_lo(i0, c), WIN)], wins[b], csems[b]).wait()
        pltpu.make_async_copy(
            x_hbm.at[pl.ds(pl.multiple_of(c * W, W), W)], xbs[b], csems[b]).wait()

    def out_block(c):
        return out_hbm.at[pl.ds(pl.multiple_of(i0, RW), RW),
                          pl.ds(pl.multiple_of(c * W, W), W)]

    issue_copies(0, 0)
    issue_copies(1, 1)

    def chunk_pair(c2, carry):
        for b in range(2):
            c = c2 * 2 + b
            wait_copies(c, b)

            @pl.when(c2 >= 1)
            def _():
                pltpu.make_async_copy(ress[b], out_block(c - 2), ssems[b]).wait()

            win, xb, res = wins[b], xbs[b], ress[b]

            def col(m, inner):
                xrow = [xb[m, pl.ds(k * L, L)] for k in range(D // L)]
                for i_r in range(RW):
                    o = (RW - 1) + i_r - m
                    for k in range(D // L):
                        sl = pl.ds(k * L, L)
                        res[i_r, m, sl] = xrow[k] + win[o, sl]
                return inner

            lax.fori_loop(0, W, col, 0)

            pltpu.make_async_copy(res, out_block(c), ssems[b]).start()

            @pl.when(c2 < NCH // 2 - 1)
            def _():
                issue_copies(c + 2, b)
        return carry

    lax.fori_loop(0, NCH // 2, chunk_pair, 0)

    pltpu.make_async_copy(ress[0], out_block(NCH - 2), ssems[0]).wait()
    pltpu.make_async_copy(ress[1], out_block(NCH - 1), ssems[1]).wait()


_sc_call = functools.partial(
    pl.kernel,
    mesh=plsc.VectorSubcoreMesh(core_axis_name="c", subcore_axis_name="s"),
    out_type=jax.ShapeDtypeStruct((S, S, D), jnp.float32),
    scratch_types=[
        pltpu.VMEM((WIN, D), jnp.float32),
        pltpu.VMEM((WIN, D), jnp.float32),
        pltpu.VMEM((W, D), jnp.float32),
        pltpu.VMEM((W, D), jnp.float32),
        pltpu.VMEM((RW, W, D), jnp.float32),
        pltpu.VMEM((RW, W, D), jnp.float32),
        pltpu.SemaphoreType.DMA,
        pltpu.SemaphoreType.DMA,
        pltpu.SemaphoreType.DMA,
        pltpu.SemaphoreType.DMA,
    ],
)(_sc_body)


def kernel(x, emb_table):
    emb_pad = jnp.concatenate(
        [emb_table, jnp.zeros((1, D), emb_table.dtype)], axis=0)
    return _sc_call(emb_pad, x[0])


# speedup vs baseline: 1.4522x; 1.4522x over previous
import functools

import jax
import jax.numpy as jnp
from jax import lax
from jax.experimental import pallas as pl
from jax.experimental.pallas import tpu as pltpu
from jax.experimental.pallas import tpu_sc as plsc

S = 512
D = 128
T = 2 * S - 1
NC = 2
NS = 16
NW = NC * NS
RW = S // NW
W = 16
NCH = S // W
WIN = W + RW
L = 16


def _win_lo(i0, c):
    return pl.multiple_of(i0 + (S - 1) - c * W - (W - 1), W)


def _sc_body(emb_hbm, x_hbm, out_hbm,
             win0, win1, xb0, xb1, res0, res1,
             csem0, csem1, ssem0, ssem1):
    wid = lax.axis_index("s") * NC + lax.axis_index("c")
    i0 = wid * RW
    wins = (win0, win1)
    xbs = (xb0, xb1)
    ress = (res0, res1)
    csems = (csem0, csem1)
    ssems = (ssem0, ssem1)

    def issue_copies(c, b):
        pltpu.make_async_copy(
            emb_hbm.at[pl.ds(_win_lo(i0, c), WIN)], wins[b], csems[b]).start()
        pltpu.make_async_copy(
            x_hbm.at[pl.ds(pl.multiple_of(c * W, W), W)], xbs[b], csems[b]).start()

    def wait_copies(c, b):
        pltpu.make_async_copy(
            emb_hbm.at[pl.ds(_win_lo(i0, c), WIN)], wins[b], csems[b]).wait()
        pltpu.make_async_copy(
            x_hbm.at[pl.ds(pl.multiple_of(c * W, W), W)], xbs[b], csems[b]).wait()

    def out_block(c):
        return out_hbm.at[pl.ds(pl.multiple_of(i0, RW), RW),
                          pl.ds(pl.multiple_of(c * W, W), W)]

    issue_copies(0, 0)
    issue_copies(1, 1)

    def chunk_pair(c2, carry):
        for b in range(2):
            c = c2 * 2 + b
            wait_copies(c, b)

            @pl.when(c2 >= 1)
            def _():
                pltpu.make_async_copy(ress[b], out_block(c - 2), ssems[b]).wait()

            win, xb, res = wins[b], xbs[b], ress[b]

            @plsc.parallel_loop(0, W, unroll=2)
            def col(m):
                xrow = [xb[m, pl.ds(k * L, L)] for k in range(D // L)]
                for i_r in range(RW):
                    o = (RW - 1) + i_r - m
                    for k in range(D // L):
                        sl = pl.ds(k * L, L)
                        res[i_r, m, sl] = xrow[k] + win[o, sl]

            pltpu.make_async_copy(res, out_block(c), ssems[b]).start()

            @pl.when(c2 < NCH // 2 - 1)
            def _():
                issue_copies(c + 2, b)
        return carry

    lax.fori_loop(0, NCH // 2, chunk_pair, 0)

    pltpu.make_async_copy(ress[0], out_block(NCH - 2), ssems[0]).wait()
    pltpu.make_async_copy(ress[1], out_block(NCH - 1), ssems[1]).wait()


_sc_call = functools.partial(
    pl.kernel,
    mesh=plsc.VectorSubcoreMesh(core_axis_name="c", subcore_axis_name="s"),
    out_type=jax.ShapeDtypeStruct((S, S, D), jnp.float32),
    scratch_types=[
        pltpu.VMEM((WIN, D), jnp.float32),
        pltpu.VMEM((WIN, D), jnp.float32),
        pltpu.VMEM((W, D), jnp.float32),
        pltpu.VMEM((W, D), jnp.float32),
        pltpu.VMEM((RW, W, D), jnp.float32),
        pltpu.VMEM((RW, W, D), jnp.float32),
        pltpu.SemaphoreType.DMA,
        pltpu.SemaphoreType.DMA,
        pltpu.SemaphoreType.DMA,
        pltpu.SemaphoreType.DMA,
    ],
)(_sc_body)


def kernel(x, emb_table):
    emb_pad = jnp.concatenate(
        [emb_table, jnp.zeros((1, D), emb_table.dtype)], axis=0)
    return _sc_call(emb_pad, x[0])


# speedup vs baseline: 2.2022x; 1.5165x over previous
import functools

import jax
import jax.numpy as jnp
from jax import lax
from jax.experimental import pallas as pl
from jax.experimental.pallas import tpu as pltpu
from jax.experimental.pallas import tpu_sc as plsc

S = 512
D = 128
T = 2 * S - 1
NC = 2
NS = 16
NW = NC * NS
RW = S // NW
W = 8
NCH = S // W
WIN = W + RW
L = 16


def _win_lo(i0, c):
    return pl.multiple_of(i0 + (S - 1) - c * W - (W - 1), W)


def _sc_body(emb_hbm, x_hbm, out_hbm,
             win0, win1, xb0, xb1, res0, res1,
             csem0, csem1, ssem0, ssem1):
    wid = lax.axis_index("s") * NC + lax.axis_index("c")
    i0 = wid * RW
    wins = (win0, win1)
    xbs = (xb0, xb1)
    ress = (res0, res1)
    csems = (csem0, csem1)
    ssems = (ssem0, ssem1)

    def issue_copies(c, b):
        pltpu.make_async_copy(
            emb_hbm.at[pl.ds(_win_lo(i0, c), WIN)], wins[b], csems[b]).start()
        pltpu.make_async_copy(
            x_hbm.at[pl.ds(pl.multiple_of(c * W, W), W)], xbs[b], csems[b]).start()

    def wait_copies(c, b):
        pltpu.make_async_copy(
            emb_hbm.at[pl.ds(_win_lo(i0, c), WIN)], wins[b], csems[b]).wait()
        pltpu.make_async_copy(
            x_hbm.at[pl.ds(pl.multiple_of(c * W, W), W)], xbs[b], csems[b]).wait()

    def out_block(c):
        return out_hbm.at[pl.ds(pl.multiple_of(i0, RW), RW),
                          pl.ds(pl.multiple_of(c * W, W), W)]

    issue_copies(0, 0)
    issue_copies(1, 1)

    def chunk_pair(c2, carry):
        for b in range(2):
            c = c2 * 2 + b
            wait_copies(c, b)

            @pl.when(c2 >= 1)
            def _():
                pltpu.make_async_copy(ress[b], out_block(c - 2), ssems[b]).wait()

            win, xb, res = wins[b], xbs[b], ress[b]

            for m in range(W):
                xrow = [xb[m, pl.ds(k * L, L)] for k in range(D // L)]
                for i_r in range(RW):
                    o = (W - 1) + i_r - m
                    for k in range(D // L):
                        sl = pl.ds(k * L, L)
                        res[i_r, m, sl] = xrow[k] + win[o, sl]

            pltpu.make_async_copy(res, out_block(c), ssems[b]).start()

            @pl.when(c2 < NCH // 2 - 1)
            def _():
                issue_copies(c + 2, b)
        return carry

    lax.fori_loop(0, NCH // 2, chunk_pair, 0)

    pltpu.make_async_copy(ress[0], out_block(NCH - 2), ssems[0]).wait()
    pltpu.make_async_copy(ress[1], out_block(NCH - 1), ssems[1]).wait()


_sc_call = functools.partial(
    pl.kernel,
    mesh=plsc.VectorSubcoreMesh(core_axis_name="c", subcore_axis_name="s"),
    out_type=jax.ShapeDtypeStruct((S, S, D), jnp.float32),
    scratch_types=[
        pltpu.VMEM((WIN, D), jnp.float32),
        pltpu.VMEM((WIN, D), jnp.float32),
        pltpu.VMEM((W, D), jnp.float32),
        pltpu.VMEM((W, D), jnp.float32),
        pltpu.VMEM((RW, W, D), jnp.float32),
        pltpu.VMEM((RW, W, D), jnp.float32),
        pltpu.SemaphoreType.DMA,
        pltpu.SemaphoreType.DMA,
        pltpu.SemaphoreType.DMA,
        pltpu.SemaphoreType.DMA,
    ],
)(_sc_body)


def kernel(x, emb_table):
    emb_pad = jnp.concatenate(
        [emb_table, jnp.zeros((1, D), emb_table.dtype)], axis=0)
    return _sc_call(emb_pad, x[0])
